# Initial kernel scaffold; baseline (speedup 1.0000x reference)
import functools

import jax
import jax.numpy as jnp
from jax import lax
from jax.experimental import pallas as pl
from jax.experimental.pallas import tpu as pltpu
from jax.experimental.pallas import tpu_sc as plsc

N = 10000
NPAD = 10240
D = 128
E = 320000
NC = 2
NS = 16
NW = NC * NS
EPT = E // NW
ROWS_PER_SUB = NPAD // NS

_MESH = dict(core_axis_name="c", subcore_axis_name="s")


def _deg_sc(dst, w):
    KA = 2000

    @functools.partial(
        pl.kernel,
        out_type=jax.ShapeDtypeStruct((NC, NPAD), jnp.float32),
        mesh=plsc.VectorSubcoreMesh(**_MESH),
        scratch_types=[
            pltpu.VMEM((KA,), jnp.int32),
            pltpu.VMEM((KA,), jnp.float32),
            pltpu.VMEM((ROWS_PER_SUB,), jnp.float32),
            pltpu.VMEM_SHARED((NPAD,), jnp.float32),
        ],
    )
    def k(dst_hbm, w_hbm, deg_hbm, dsti, wv, initv, deg_sh):
        c = lax.axis_index("c")
        s = lax.axis_index("s")
        tile = s * NC + c

        half = jnp.full((16,), 0.5, jnp.float32)

        @pl.loop(0, ROWS_PER_SUB, step=16)
        def _(i):
            initv[pl.ds(i, 16)] = half

        pltpu.sync_copy(initv, deg_sh.at[pl.ds(s * ROWS_PER_SUB, ROWS_PER_SUB)])
        plsc.subcore_barrier()

        base = tile * EPT

        @pl.loop(0, EPT, step=KA)
        def _(off):
            pltpu.sync_copy(dst_hbm.at[pl.ds(base + off, KA)], dsti)
            pltpu.sync_copy(w_hbm.at[pl.ds(base + off, KA)], wv)
            pltpu.sync_copy(wv, deg_sh.at[dsti], add=True)

        plsc.subcore_barrier()
        pltpu.sync_copy(
            deg_sh.at[pl.ds(s * ROWS_PER_SUB, ROWS_PER_SUB)],
            deg_hbm.at[c, pl.ds(s * ROWS_PER_SUB, ROWS_PER_SUB)],
        )

    return k(dst, w)


def _tc_front(x_pad, W_gcn, deg2col):
    BR = 2048

    def body(x_ref, w_ref, deg_ref, y_ref, dinv_ref):
        degt = deg_ref[0] + deg_ref[1]
        dinv = lax.rsqrt(degt)
        dinv_ref[...] = dinv
        xw = jnp.dot(x_ref[...], w_ref[...], preferred_element_type=jnp.float32)
        y_ref[...] = xw * dinv

    return pl.pallas_call(
        body,
        grid=(NPAD // BR,),
        in_specs=[
            pl.BlockSpec((BR, D), lambda i: (i, 0)),
            pl.BlockSpec((D, D), lambda i: (0, 0)),
            pl.BlockSpec((NC, BR, 1), lambda i: (0, i, 0)),
        ],
        out_specs=[
            pl.BlockSpec((BR, D), lambda i: (i, 0)),
            pl.BlockSpec((BR, 1), lambda i: (i, 0)),
        ],
        out_shape=[
            jax.ShapeDtypeStruct((NPAD, D), jnp.float32),
            jax.ShapeDtypeStruct((NPAD, 1), jnp.float32),
        ],
    )(x_pad, W_gcn, deg2col)


def _z_sc(y, src, dst, w):
    K = 400

    @functools.partial(
        pl.kernel,
        out_type=jax.ShapeDtypeStruct((NC, NPAD, D), jnp.float32),
        mesh=plsc.VectorSubcoreMesh(**_MESH),
        scratch_types=[
            pltpu.VMEM((K,), jnp.int32),
            pltpu.VMEM((K,), jnp.int32),
            pltpu.VMEM((K,), jnp.float32),
            pltpu.VMEM((K, D), jnp.float32),
            pltpu.VMEM_SHARED((NPAD, D), jnp.float32),
        ],
    )
    def k(y_hbm, src_hbm, dst_hbm, w_hbm, z_hbm, srci, dsti, wv, rows, z_sh):
        c = lax.axis_index("c")
        s = lax.axis_index("s")
        tile = s * NC + c

        zero = jnp.zeros((16,), jnp.float32)

        @pl.loop(0, K)
        def _(e):
            for j in range(8):
                rows[e, pl.ds(j * 16, 16)] = zero

        rbase = s * ROWS_PER_SUB
        pltpu.sync_copy(rows, z_sh.at[pl.ds(rbase, K)])
        pltpu.sync_copy(
            rows.at[pl.ds(0, ROWS_PER_SUB - K)],
            z_sh.at[pl.ds(rbase + K, ROWS_PER_SUB - K)],
        )
        plsc.subcore_barrier()

        base = tile * EPT

        @pl.loop(0, EPT, step=K)
        def _(off):
            pltpu.sync_copy(src_hbm.at[pl.ds(base + off, K)], srci)
            pltpu.sync_copy(dst_hbm.at[pl.ds(base + off, K)], dsti)
            pltpu.sync_copy(w_hbm.at[pl.ds(base + off, K)], wv)
            pltpu.sync_copy(y_hbm.at[srci], rows)

            @pl.loop(0, K)
            def _(e):
                eidx = jnp.full((16,), e, jnp.int32)
                wsplat = plsc.load_gather(wv, [eidx])
                for j in range(8):
                    sl = pl.ds(j * 16, 16)
                    rows[e, sl] = rows[e, sl] * wsplat

            pltpu.sync_copy(rows, z_sh.at[dsti], add=True)

        plsc.subcore_barrier()
        pltpu.sync_copy(
            z_sh.at[pl.ds(rbase, ROWS_PER_SUB)],
            z_hbm.at[c, pl.ds(rbase, ROWS_PER_SUB)],
        )

    return k(y, src, dst, w)


def _tc_back(z, y, dinv, b_gcn, W_lin, b_lin):
    BR = 2048

    def body(z_ref, y_ref, dinv_ref, bg_ref, wl_ref, bl_ref, out_ref):
        h = (z_ref[0] + z_ref[1] + y_ref[...]) * dinv_ref[...] + bg_ref[...]
        h = jnp.maximum(h, 0.0)
        out_ref[...] = (
            jnp.dot(h, wl_ref[...], preferred_element_type=jnp.float32)
            + bl_ref[...]
        )

    return pl.pallas_call(
        body,
        grid=(NPAD // BR,),
        in_specs=[
            pl.BlockSpec((NC, BR, D), lambda i: (0, i, 0)),
            pl.BlockSpec((BR, D), lambda i: (i, 0)),
            pl.BlockSpec((BR, 1), lambda i: (i, 0)),
            pl.BlockSpec((1, D), lambda i: (0, 0)),
            pl.BlockSpec((D, 1), lambda i: (0, 0)),
            pl.BlockSpec((1, 1), lambda i: (0, 0)),
        ],
        out_specs=pl.BlockSpec((BR, 1), lambda i: (i, 0)),
        out_shape=jax.ShapeDtypeStruct((NPAD, 1), jnp.float32),
    )(z, y, dinv, b_gcn, W_lin, b_lin)


def kernel(x, edge_index, edge_weight, W_gcn, b_gcn, W_lin, b_lin):
    src = edge_index[0].astype(jnp.int32)
    dst = edge_index[1].astype(jnp.int32)
    w = edge_weight.astype(jnp.float32)

    deg2 = _deg_sc(dst, w)
    x_pad = jnp.pad(x, ((0, NPAD - N), (0, 0)))
    y, dinv = _tc_front(x_pad, W_gcn, deg2.reshape(NC, NPAD, 1))
    z = _z_sc(y, src, dst, w)
    out_pad = _tc_back(
        z, y, dinv, b_gcn.reshape(1, D), W_lin, b_lin.reshape(1, 1)
    )
    return out_pad[:N]


# speedup vs baseline: 19.5241x; 19.5241x over previous
import dataclasses
import functools

import jax
import jax.numpy as jnp
from jax import lax
from jax.experimental import pallas as pl
from jax.experimental.pallas import tpu as pltpu
from jax.experimental.pallas import tpu_sc as plsc

N = 10000
NPAD = 10240
D = 128
E = 320000
NC = 2
NS = 16
NW = NC * NS
EPT = E // NW
ROWS_PER_SUB = NPAD // NS

_MESH = dict(core_axis_name="c", subcore_axis_name="s")


def _sc_params():
    cp = pltpu.CompilerParams()
    if "needs_layout_passes" in pltpu.CompilerParams.__dataclass_fields__:
        cp = dataclasses.replace(cp, needs_layout_passes=False)
    return cp


def _deg_sc(dst, w):
    KA = 2000

    @functools.partial(
        pl.kernel,
        out_type=jax.ShapeDtypeStruct((NC, NPAD), jnp.float32),
        mesh=plsc.VectorSubcoreMesh(**_MESH),
        scratch_types=[
            pltpu.VMEM((KA,), jnp.int32),
            pltpu.VMEM((KA,), jnp.float32),
            pltpu.VMEM((ROWS_PER_SUB,), jnp.float32),
            pltpu.VMEM_SHARED((NPAD,), jnp.float32),
        ],
    )
    def k(dst_hbm, w_hbm, deg_hbm, dsti, wv, initv, deg_sh):
        c = lax.axis_index("c")
        s = lax.axis_index("s")
        tile = s * NC + c

        half = jnp.full((16,), 0.5, jnp.float32)

        @pl.loop(0, ROWS_PER_SUB, step=16)
        def _(i):
            initv[pl.ds(i, 16)] = half

        pltpu.sync_copy(initv, deg_sh.at[pl.ds(s * ROWS_PER_SUB, ROWS_PER_SUB)])
        plsc.subcore_barrier()

        base = tile * EPT

        @pl.loop(0, EPT, step=KA)
        def _(off):
            pltpu.sync_copy(dst_hbm.at[pl.ds(base + off, KA)], dsti)
            pltpu.sync_copy(w_hbm.at[pl.ds(base + off, KA)], wv)
            pltpu.sync_copy(wv, deg_sh.at[dsti], add=True)

        plsc.subcore_barrier()
        pltpu.sync_copy(
            deg_sh.at[pl.ds(s * ROWS_PER_SUB, ROWS_PER_SUB)],
            deg_hbm.at[c, pl.ds(s * ROWS_PER_SUB, ROWS_PER_SUB)],
        )

    return k(dst, w)


def _tc_front(x_pad, W_gcn, deg2col):
    BR = 2048

    def body(x_ref, w_ref, deg_ref, y_ref, dinv_ref):
        degt = deg_ref[0] + deg_ref[1]
        dinv = lax.rsqrt(degt)
        dinv_ref[...] = dinv
        xw = jnp.dot(x_ref[...], w_ref[...], preferred_element_type=jnp.float32)
        y_ref[...] = xw * dinv

    return pl.pallas_call(
        body,
        grid=(NPAD // BR,),
        in_specs=[
            pl.BlockSpec((BR, D), lambda i: (i, 0)),
            pl.BlockSpec((D, D), lambda i: (0, 0)),
            pl.BlockSpec((NC, BR, 1), lambda i: (0, i, 0)),
        ],
        out_specs=[
            pl.BlockSpec((BR, D), lambda i: (i, 0)),
            pl.BlockSpec((BR, 1), lambda i: (i, 0)),
        ],
        out_shape=[
            jax.ShapeDtypeStruct((NPAD, D), jnp.float32),
            jax.ShapeDtypeStruct((NPAD, 1), jnp.float32),
        ],
    )(x_pad, W_gcn, deg2col)


def _z_sc(y, src, dst, w):
    K = 200

    @functools.partial(
        pl.kernel,
        out_type=jax.ShapeDtypeStruct((NC, NPAD, D), jnp.float32),
        mesh=plsc.VectorSubcoreMesh(**_MESH),
        compiler_params=_sc_params(),
        scratch_types=[
            pltpu.VMEM((K,), jnp.int32),
            pltpu.VMEM((K,), jnp.int32),
            pltpu.VMEM((K,), jnp.float32),
            pltpu.VMEM((K, D), jnp.float32),
            pltpu.VMEM_SHARED((NPAD, D), jnp.float32),
        ],
    )
    def k(y_hbm, src_hbm, dst_hbm, w_hbm, z_hbm, srci, dsti, wv, rows, z_sh):
        c = lax.axis_index("c")
        s = lax.axis_index("s")
        tile = s * NC + c

        zero = jnp.zeros((16,), jnp.float32)

        @pl.loop(0, K)
        def _(e):
            for j in range(8):
                rows[e, pl.ds(j * 16, 16)] = zero

        rbase = s * ROWS_PER_SUB
        for t in range(ROWS_PER_SUB // 160):
            pltpu.sync_copy(
                rows.at[pl.ds(0, 160)],
                z_sh.at[pl.ds(rbase + 160 * t, 160)],
            )
        plsc.subcore_barrier()

        base = tile * EPT

        @pl.loop(0, EPT, step=K)
        def _(off):
            pltpu.sync_copy(src_hbm.at[pl.ds(base + off, K)], srci)
            pltpu.sync_copy(dst_hbm.at[pl.ds(base + off, K)], dsti)
            pltpu.sync_copy(w_hbm.at[pl.ds(base + off, K)], wv)
            pltpu.sync_copy(y_hbm.at[srci], rows)

            @pl.loop(0, K)
            def _(e):
                eidx = jnp.full((16,), e, jnp.int32)
                wsplat = plsc.load_gather(wv, [eidx])
                for j in range(8):
                    sl = pl.ds(j * 16, 16)
                    rows[e, sl] = rows[e, sl] * wsplat

            pltpu.sync_copy(rows, z_sh.at[dsti], add=True)

        plsc.subcore_barrier()
        pltpu.sync_copy(
            z_sh.at[pl.ds(rbase, ROWS_PER_SUB)],
            z_hbm.at[c, pl.ds(rbase, ROWS_PER_SUB)],
        )

    return k(y, src, dst, w)


def _tc_back(z, y, dinv, b_gcn, W_lin, b_lin):
    BR = 2048

    def body(z_ref, y_ref, dinv_ref, bg_ref, wl_ref, bl_ref, out_ref):
        h = (z_ref[0] + z_ref[1] + y_ref[...]) * dinv_ref[...] + bg_ref[...]
        h = jnp.maximum(h, 0.0)
        out_ref[...] = (
            jnp.dot(h, wl_ref[...], preferred_element_type=jnp.float32)
            + bl_ref[...]
        )

    return pl.pallas_call(
        body,
        grid=(NPAD // BR,),
        in_specs=[
            pl.BlockSpec((NC, BR, D), lambda i: (0, i, 0)),
            pl.BlockSpec((BR, D), lambda i: (i, 0)),
            pl.BlockSpec((BR, 1), lambda i: (i, 0)),
            pl.BlockSpec((1, D), lambda i: (0, 0)),
            pl.BlockSpec((D, 1), lambda i: (0, 0)),
            pl.BlockSpec((1, 1), lambda i: (0, 0)),
        ],
        out_specs=pl.BlockSpec((BR, 1), lambda i: (i, 0)),
        out_shape=jax.ShapeDtypeStruct((NPAD, 1), jnp.float32),
    )(z, y, dinv, b_gcn, W_lin, b_lin)


def kernel(x, edge_index, edge_weight, W_gcn, b_gcn, W_lin, b_lin):
    src = edge_index[0].astype(jnp.int32)
    dst = edge_index[1].astype(jnp.int32)
    w = edge_weight.astype(jnp.float32)

    deg2 = _deg_sc(dst, w)
    x_pad = jnp.pad(x, ((0, NPAD - N), (0, 0)))
    y, dinv = _tc_front(x_pad, W_gcn, deg2.reshape(NC, NPAD, 1))
    z = _z_sc(y, src, dst, w)
    out_pad = _tc_back(
        z, y, dinv, b_gcn.reshape(1, D), W_lin, b_lin.reshape(1, 1)
    )
    return out_pad[:N]


# speedup vs baseline: 33.0712x; 1.6939x over previous
import dataclasses
import functools

import jax
import jax.numpy as jnp
from jax import lax
from jax.experimental import pallas as pl
from jax.experimental.pallas import tpu as pltpu
from jax.experimental.pallas import tpu_sc as plsc

N = 10000
NPAD = 10240
D = 128
E = 320000
NC = 2
NS = 16
NW = NC * NS
EPT = E // NW
ROWS_PER_SUB = NPAD // NS

_MESH = dict(core_axis_name="c", subcore_axis_name="s")


def _sc_params():
    cp = pltpu.CompilerParams()
    if "needs_layout_passes" in pltpu.CompilerParams.__dataclass_fields__:
        cp = dataclasses.replace(cp, needs_layout_passes=False)
    return cp


def _deg_sc(dst, w):
    KA = 2000

    @functools.partial(
        pl.kernel,
        out_type=jax.ShapeDtypeStruct((NC, NPAD), jnp.float32),
        mesh=plsc.VectorSubcoreMesh(**_MESH),
        scratch_types=[
            pltpu.VMEM((KA,), jnp.int32),
            pltpu.VMEM((KA,), jnp.float32),
            pltpu.VMEM((ROWS_PER_SUB,), jnp.float32),
            pltpu.VMEM_SHARED((NPAD,), jnp.float32),
        ],
    )
    def k(dst_hbm, w_hbm, deg_hbm, dsti, wv, initv, deg_sh):
        c = lax.axis_index("c")
        s = lax.axis_index("s")
        tile = s * NC + c

        half = jnp.full((16,), 0.5, jnp.float32)

        @pl.loop(0, ROWS_PER_SUB, step=16)
        def _(i):
            initv[pl.ds(i, 16)] = half

        pltpu.sync_copy(initv, deg_sh.at[pl.ds(s * ROWS_PER_SUB, ROWS_PER_SUB)])
        plsc.subcore_barrier()

        base = tile * EPT

        @pl.loop(0, EPT, step=KA)
        def _(off):
            pltpu.sync_copy(dst_hbm.at[pl.ds(base + off, KA)], dsti)
            pltpu.sync_copy(w_hbm.at[pl.ds(base + off, KA)], wv)
            pltpu.sync_copy(wv, deg_sh.at[dsti], add=True)

        plsc.subcore_barrier()
        pltpu.sync_copy(
            deg_sh.at[pl.ds(s * ROWS_PER_SUB, ROWS_PER_SUB)],
            deg_hbm.at[c, pl.ds(s * ROWS_PER_SUB, ROWS_PER_SUB)],
        )

    return k(dst, w)


def _tc_front(x, W_gcn, deg2col):
    BR = 2000

    def body(x_ref, w_ref, deg_ref, y_ref, dinv_ref):
        degt = deg_ref[0] + deg_ref[1]
        dinv = lax.rsqrt(degt)
        dinv_ref[...] = dinv
        xw = jnp.dot(x_ref[...], w_ref[...], preferred_element_type=jnp.float32)
        y_ref[...] = xw * dinv

    return pl.pallas_call(
        body,
        grid=(N // BR,),
        in_specs=[
            pl.BlockSpec((BR, D), lambda i: (i, 0)),
            pl.BlockSpec((D, D), lambda i: (0, 0)),
            pl.BlockSpec((NC, BR, 1), lambda i: (0, i, 0)),
        ],
        out_specs=[
            pl.BlockSpec((BR, D), lambda i: (i, 0)),
            pl.BlockSpec((BR, 1), lambda i: (i, 0)),
        ],
        out_shape=[
            jax.ShapeDtypeStruct((N, D), jnp.float32),
            jax.ShapeDtypeStruct((N, 1), jnp.float32),
        ],
    )(x, W_gcn, deg2col)


K = 125
NCH = EPT // K
NB = 3
SLICE = 624


def _z_sc(y, src3, dst3, w3):
    bufs = []
    for _ in range(NB):
        bufs += [
            pltpu.VMEM((1, K), jnp.int32),
            pltpu.VMEM((1, K), jnp.int32),
            pltpu.VMEM((1, K), jnp.float32),
            pltpu.VMEM((K, D), jnp.float32),
            pltpu.SemaphoreType.DMA,
            pltpu.SemaphoreType.DMA,
            pltpu.SemaphoreType.DMA,
        ]

    @functools.partial(
        pl.kernel,
        out_type=jax.ShapeDtypeStruct((NC, N, D), jnp.float32),
        mesh=plsc.VectorSubcoreMesh(**_MESH),
        compiler_params=_sc_params(),
        scratch_types=bufs + [
            pltpu.VMEM_SHARED((N, D), jnp.float32),
        ],
    )
    def k(y_hbm, src_hbm, dst_hbm, w_hbm, z_hbm, *scr):
        z_sh = scr[-1]
        srcis = tuple(scr[7 * b + 0] for b in range(NB))
        dstis = tuple(scr[7 * b + 1] for b in range(NB))
        wvas = tuple(scr[7 * b + 2] for b in range(NB))
        rowss = tuple(scr[7 * b + 3] for b in range(NB))
        semi = tuple(scr[7 * b + 4] for b in range(NB))
        semg = tuple(scr[7 * b + 5] for b in range(NB))
        sems = tuple(scr[7 * b + 6] for b in range(NB))

        c = lax.axis_index("c")
        s = lax.axis_index("s")
        tile = s * NC + c

        zero16 = jnp.zeros((16,), jnp.float32)
        rows0 = rowss[0]

        @pl.loop(0, K)
        def _(e):
            for j in range(8):
                rows0[e, pl.ds(j * 16, 16)] = zero16

        rbase = s * SLICE
        for t in range(SLICE // 120):
            pltpu.sync_copy(
                rows0.at[pl.ds(0, 120)],
                z_sh.at[pl.ds(rbase + 120 * t, 120)],
            )
        pltpu.sync_copy(
            rows0.at[pl.ds(0, 24)],
            z_sh.at[pl.ds(rbase + 600, 24)],
        )

        @pl.when(s == 0)
        def _():
            pltpu.sync_copy(
                rows0.at[pl.ds(0, N - NS * SLICE)],
                z_sh.at[pl.ds(NS * SLICE, N - NS * SLICE)],
            )

        plsc.subcore_barrier()

        def idx_copies(ch, b):
            return (
                pltpu.make_async_copy(
                    src_hbm.at[tile, pl.ds(ch, 1)], srcis[b], semi[b]),
                pltpu.make_async_copy(
                    dst_hbm.at[tile, pl.ds(ch, 1)], dstis[b], semi[b]),
                pltpu.make_async_copy(
                    w_hbm.at[tile, pl.ds(ch, 1)], wvas[b], semi[b]),
            )

        def issue_idx(ch, b):
            for cp in idx_copies(ch, b):
                cp.start()

        def wait_idx(ch, b):
            for cp in idx_copies(ch, b):
                cp.wait()

        def gather_copy(ch, b):
            return pltpu.make_async_copy(
                y_hbm.at[srcis[b].at[0]], rowss[b], semg[b])

        def scatter_copy(ch, b):
            return pltpu.make_async_copy(
                rowss[b], z_sh.at[dstis[b].at[0]], sems[b])

        def scale(b):
            wbuf = wvas[b]
            rbuf = rowss[b]
            zrow = jnp.zeros((16,), jnp.int32)

            @pl.loop(0, K, step=5)
            def _(ei):
                for u in range(5):
                    e = ei + u
                    eidx = jnp.full((16,), e, jnp.int32)
                    wsp = plsc.load_gather(wbuf, [zrow, eidx])
                    for j in range(8):
                        sl = pl.ds(j * 16, 16)
                        rbuf[e, sl] = rbuf[e, sl] * wsp

        issue_idx(0, 0)
        issue_idx(1, 1)
        wait_idx(0, 0)
        gather_copy(0, 0).start()

        @pl.loop(0, NCH - 2, step=NB)
        def _(g):
            for b in range(NB):
                ch = g + b
                b_next = (b + 1) % NB
                b_prev = (b + 2) % NB

                @pl.when(ch >= 1)
                def _():
                    scatter_copy(ch - 1, b_prev).wait()

                issue_idx(ch + 2, b_prev)
                wait_idx(ch + 1, b_next)
                gather_copy(ch + 1, b_next).start()
                gather_copy(ch, b).wait()
                scale(b)
                scatter_copy(ch, b).start(add=True)

        scatter_copy(NCH - 3, 2).wait()
        wait_idx(NCH - 1, 1)
        gather_copy(NCH - 1, 1).start()
        gather_copy(NCH - 2, 0).wait()
        scale(0)
        scatter_copy(NCH - 2, 0).start(add=True)
        scatter_copy(NCH - 2, 0).wait()
        gather_copy(NCH - 1, 1).wait()
        scale(1)
        scatter_copy(NCH - 1, 1).start(add=True)
        scatter_copy(NCH - 1, 1).wait()
        plsc.subcore_barrier()

        pltpu.sync_copy(
            z_sh.at[pl.ds(rbase, SLICE)],
            z_hbm.at[c, pl.ds(rbase, SLICE)],
        )

        @pl.when(s == 0)
        def _():
            pltpu.sync_copy(
                z_sh.at[pl.ds(NS * SLICE, N - NS * SLICE)],
                z_hbm.at[c, pl.ds(NS * SLICE, N - NS * SLICE)],
            )

    return k(y, src3, dst3, w3)


def _tc_back(z, y, dinv, b_gcn, W_lin, b_lin):
    BR = 2000

    def body(z_ref, y_ref, dinv_ref, bg_ref, wl_ref, bl_ref, out_ref):
        zs = (z_ref[0].astype(jnp.float32) + z_ref[1].astype(jnp.float32)
              + y_ref[...].astype(jnp.float32))
        h = zs * dinv_ref[...] + bg_ref[...]
        h = jnp.maximum(h, 0.0)
        out_ref[...] = (
            jnp.dot(h, wl_ref[...], preferred_element_type=jnp.float32)
            + bl_ref[...]
        )

    return pl.pallas_call(
        body,
        grid=(N // BR,),
        in_specs=[
            pl.BlockSpec((NC, BR, D), lambda i: (0, i, 0)),
            pl.BlockSpec((BR, D), lambda i: (i, 0)),
            pl.BlockSpec((BR, 1), lambda i: (i, 0)),
            pl.BlockSpec((1, D), lambda i: (0, 0)),
            pl.BlockSpec((D, 1), lambda i: (0, 0)),
            pl.BlockSpec((1, 1), lambda i: (0, 0)),
        ],
        out_specs=pl.BlockSpec((BR, 1), lambda i: (i, 0)),
        out_shape=jax.ShapeDtypeStruct((N, 1), jnp.float32),
    )(z, y, dinv, b_gcn, W_lin, b_lin)


def kernel(x, edge_index, edge_weight, W_gcn, b_gcn, W_lin, b_lin):
    src = edge_index[0].astype(jnp.int32)
    dst = edge_index[1].astype(jnp.int32)
    w = edge_weight.astype(jnp.float32)

    deg2 = _deg_sc(dst, w)
    y, dinv = _tc_front(x, W_gcn, deg2[:, :N].reshape(NC, N, 1))
    z = _z_sc(
        y,
        src.reshape(NW, NCH, K),
        dst.reshape(NW, NCH, K),
        w.reshape(NW, NCH, K),
    )
    return _tc_back(
        z, y, dinv, b_gcn.reshape(1, D), W_lin, b_lin.reshape(1, 1)
    )


# speedup vs baseline: 38.4770x; 1.1635x over previous
import dataclasses
import functools

import jax
import jax.numpy as jnp
from jax import lax
from jax.experimental import pallas as pl
from jax.experimental.pallas import tpu as pltpu
from jax.experimental.pallas import tpu_sc as plsc

N = 10000
NPAD = 10240
D = 128
E = 320000
NC = 2
NS = 16
NW = NC * NS
EPT = E // NW
ROWS_PER_SUB = NPAD // NS

_MESH = dict(core_axis_name="c", subcore_axis_name="s")


def _sc_params():
    cp = pltpu.CompilerParams()
    if "needs_layout_passes" in pltpu.CompilerParams.__dataclass_fields__:
        cp = dataclasses.replace(cp, needs_layout_passes=False)
    return cp


def _deg_sc(dst, w):
    KA = 2000

    @functools.partial(
        pl.kernel,
        out_type=jax.ShapeDtypeStruct((NC, NPAD), jnp.float32),
        mesh=plsc.VectorSubcoreMesh(**_MESH),
        scratch_types=[
            pltpu.VMEM((KA,), jnp.int32),
            pltpu.VMEM((KA,), jnp.float32),
            pltpu.VMEM((ROWS_PER_SUB,), jnp.float32),
            pltpu.VMEM_SHARED((NPAD,), jnp.float32),
        ],
    )
    def k(dst_hbm, w_hbm, deg_hbm, dsti, wv, initv, deg_sh):
        c = lax.axis_index("c")
        s = lax.axis_index("s")
        tile = s * NC + c

        half = jnp.full((16,), 0.5, jnp.float32)

        @pl.loop(0, ROWS_PER_SUB, step=16)
        def _(i):
            initv[pl.ds(i, 16)] = half

        pltpu.sync_copy(initv, deg_sh.at[pl.ds(s * ROWS_PER_SUB, ROWS_PER_SUB)])
        plsc.subcore_barrier()

        base = tile * EPT

        @pl.loop(0, EPT, step=KA)
        def _(off):
            pltpu.sync_copy(dst_hbm.at[pl.ds(base + off, KA)], dsti)
            pltpu.sync_copy(w_hbm.at[pl.ds(base + off, KA)], wv)
            pltpu.sync_copy(wv, deg_sh.at[dsti], add=True)

        plsc.subcore_barrier()
        pltpu.sync_copy(
            deg_sh.at[pl.ds(s * ROWS_PER_SUB, ROWS_PER_SUB)],
            deg_hbm.at[c, pl.ds(s * ROWS_PER_SUB, ROWS_PER_SUB)],
        )

    return k(dst, w)


def _tc_front(x, W_gcn, deg2col):
    BR = 2000

    def body(x_ref, w_ref, deg_ref, y_ref, dinv_ref):
        degt = deg_ref[0] + deg_ref[1]
        dinv = lax.rsqrt(degt)
        dinv_ref[...] = dinv
        xw = jnp.dot(x_ref[...], w_ref[...], preferred_element_type=jnp.float32)
        y_ref[...] = xw * dinv

    return pl.pallas_call(
        body,
        grid=(N // BR,),
        in_specs=[
            pl.BlockSpec((BR, D), lambda i: (i, 0)),
            pl.BlockSpec((D, D), lambda i: (0, 0)),
            pl.BlockSpec((NC, BR, 1), lambda i: (0, i, 0)),
        ],
        out_specs=[
            pl.BlockSpec((BR, D), lambda i: (i, 0)),
            pl.BlockSpec((BR, 1), lambda i: (i, 0)),
        ],
        out_shape=[
            jax.ShapeDtypeStruct((N, D), jnp.float32),
            jax.ShapeDtypeStruct((N, 1), jnp.float32),
        ],
    )(x, W_gcn, deg2col)


K = 125
NCH = EPT // K
NB = 3
ND = 6
SLICE = 624


def _z_sc(y, src3, dst3, w3):
    bbufs = []
    for _ in range(NB):
        bbufs += [
            pltpu.VMEM((1, K), jnp.int32),
            pltpu.VMEM((1, K), jnp.float32),
            pltpu.VMEM((K, D), jnp.float32),
            pltpu.SemaphoreType.DMA,
            pltpu.SemaphoreType.DMA,
            pltpu.SemaphoreType.DMA,
        ]
    dbufs = []
    for _ in range(ND):
        dbufs += [
            pltpu.VMEM((1, K), jnp.int32),
            pltpu.SemaphoreType.DMA,
        ]

    @functools.partial(
        pl.kernel,
        out_type=jax.ShapeDtypeStruct((NC, N, D), jnp.float32),
        mesh=plsc.VectorSubcoreMesh(**_MESH),
        compiler_params=_sc_params(),
        scratch_types=bbufs + dbufs + [
            pltpu.VMEM_SHARED((N, D), jnp.float32),
        ],
    )
    def k(y_hbm, src_hbm, dst_hbm, w_hbm, z_hbm, *scr):
        z_sh = scr[-1]
        srcis = tuple(scr[6 * b + 0] for b in range(NB))
        wvas = tuple(scr[6 * b + 1] for b in range(NB))
        rowss = tuple(scr[6 * b + 2] for b in range(NB))
        semw = tuple(scr[6 * b + 3] for b in range(NB))
        semg = tuple(scr[6 * b + 4] for b in range(NB))
        sems = tuple(scr[6 * b + 5] for b in range(NB))
        dof = 6 * NB
        dstis = tuple(scr[dof + 2 * i + 0] for i in range(ND))
        semd = tuple(scr[dof + 2 * i + 1] for i in range(ND))

        c = lax.axis_index("c")
        s = lax.axis_index("s")
        tile = s * NC + c

        zero16 = jnp.zeros((16,), jnp.float32)
        rows0 = rowss[0]

        @pl.loop(0, K)
        def _(e):
            for j in range(8):
                rows0[e, pl.ds(j * 16, 16)] = zero16

        rbase = s * SLICE
        for t in range(SLICE // 112):
            pltpu.sync_copy(
                rows0.at[pl.ds(0, 112)],
                z_sh.at[pl.ds(rbase + 112 * t, 112)],
            )
        pltpu.sync_copy(
            rows0.at[pl.ds(0, 64)],
            z_sh.at[pl.ds(rbase + 560, 64)],
        )

        @pl.when(s == 0)
        def _():
            pltpu.sync_copy(
                rows0.at[pl.ds(0, N - NS * SLICE)],
                z_sh.at[pl.ds(NS * SLICE, N - NS * SLICE)],
            )

        plsc.subcore_barrier()

        def sw_copies(ch, b):
            return (
                pltpu.make_async_copy(
                    src_hbm.at[tile, pl.ds(ch, 1)], srcis[b], semw[b]),
                pltpu.make_async_copy(
                    w_hbm.at[tile, pl.ds(ch, 1)], wvas[b], semw[b]),
            )

        def dst_copy(ch, i):
            return pltpu.make_async_copy(
                dst_hbm.at[tile, pl.ds(ch, 1)], dstis[i], semd[i])

        def issue_idx(ch, b, i):
            for cp in sw_copies(ch, b):
                cp.start()
            dst_copy(ch, i).start()

        def gather_copy(ch, b):
            return pltpu.make_async_copy(
                y_hbm.at[srcis[b].at[0]], rowss[b], semg[b])

        def scatter_copy(ch, i, b):
            return pltpu.make_async_copy(
                rowss[b], z_sh.at[dstis[i].at[0]], sems[b])

        def scale(b):
            wbuf = wvas[b]
            rbuf = rowss[b]
            zrow = jnp.zeros((16,), jnp.int32)

            @pl.loop(0, K, step=5)
            def _(ei):
                for u in range(5):
                    e = ei + u
                    eidx = jnp.full((16,), e, jnp.int32)
                    wsp = plsc.load_gather(wbuf, [zrow, eidx])
                    for j in range(8):
                        sl = pl.ds(j * 16, 16)
                        rbuf[e, sl] = rbuf[e, sl] * wsp

        for m in range(3):
            issue_idx(m, m, m)
        for cp in sw_copies(0, 0):
            cp.wait()
        gather_copy(0, 0).start()

        @pl.loop(0, NCH - 2, step=NB)
        def _(g):
            par = (g // NB) % 2
            for b in range(NB):
                ch = g + b
                b_next = (b + 1) % NB
                b_prev = (b + 2) % NB

                def run(d_cur, d_pre, d_fut):
                    for cp in sw_copies(ch + 1, b_next):
                        cp.wait()
                    gather_copy(ch + 1, b_next).start()
                    gather_copy(ch, b).wait()
                    scale(b)

                    @pl.when(ch >= 1)
                    def _():
                        scatter_copy(ch - 1, d_pre, b_prev).wait()

                    @pl.when(ch + 3 <= NCH - 1)
                    def _():
                        issue_idx(ch + 3, b, d_fut)

                    dst_copy(ch, d_cur).wait()
                    scatter_copy(ch, d_cur, b).start(add=True)

                @pl.when(par == 0)
                def _():
                    run(b, (b - 1) % ND, (b + 3) % ND)

                @pl.when(par == 1)
                def _():
                    run((b + 3) % ND, (b + 2) % ND, b)

        d_a = (NCH - 2) % ND
        d_b = (NCH - 1) % ND
        d_p = (NCH - 3) % ND
        for cp in sw_copies(NCH - 1, 1):
            cp.wait()
        gather_copy(NCH - 1, 1).start()
        gather_copy(NCH - 2, 0).wait()
        scale(0)
        scatter_copy(NCH - 3, d_p, 2).wait()
        dst_copy(NCH - 2, d_a).wait()
        scatter_copy(NCH - 2, d_a, 0).start(add=True)
        gather_copy(NCH - 1, 1).wait()
        scale(1)
        scatter_copy(NCH - 2, d_a, 0).wait()
        dst_copy(NCH - 1, d_b).wait()
        scatter_copy(NCH - 1, d_b, 1).start(add=True)
        scatter_copy(NCH - 1, d_b, 1).wait()
        plsc.subcore_barrier()

        pltpu.sync_copy(
            z_sh.at[pl.ds(rbase, SLICE)],
            z_hbm.at[c, pl.ds(rbase, SLICE)],
        )

        @pl.when(s == 0)
        def _():
            pltpu.sync_copy(
                z_sh.at[pl.ds(NS * SLICE, N - NS * SLICE)],
                z_hbm.at[c, pl.ds(NS * SLICE, N - NS * SLICE)],
            )

    return k(y, src3, dst3, w3)


def _tc_back(z, y, dinv, b_gcn, W_lin, b_lin):
    BR = 2000

    def body(z_ref, y_ref, dinv_ref, bg_ref, wl_ref, bl_ref, out_ref):
        zs = (z_ref[0].astype(jnp.float32) + z_ref[1].astype(jnp.float32)
              + y_ref[...].astype(jnp.float32))
        h = zs * dinv_ref[...] + bg_ref[...]
        h = jnp.maximum(h, 0.0)
        out_ref[...] = (
            jnp.dot(h, wl_ref[...], preferred_element_type=jnp.float32)
            + bl_ref[...]
        )

    return pl.pallas_call(
        body,
        grid=(N // BR,),
        in_specs=[
            pl.BlockSpec((NC, BR, D), lambda i: (0, i, 0)),
            pl.BlockSpec((BR, D), lambda i: (i, 0)),
            pl.BlockSpec((BR, 1), lambda i: (i, 0)),
            pl.BlockSpec((1, D), lambda i: (0, 0)),
            pl.BlockSpec((D, 1), lambda i: (0, 0)),
            pl.BlockSpec((1, 1), lambda i: (0, 0)),
        ],
        out_specs=pl.BlockSpec((BR, 1), lambda i: (i, 0)),
        out_shape=jax.ShapeDtypeStruct((N, 1), jnp.float32),
    )(z, y, dinv, b_gcn, W_lin, b_lin)


def kernel(x, edge_index, edge_weight, W_gcn, b_gcn, W_lin, b_lin):
    src = edge_index[0].astype(jnp.int32)
    dst = edge_index[1].astype(jnp.int32)
    w = edge_weight.astype(jnp.float32)

    deg2 = _deg_sc(dst, w)
    y, dinv = _tc_front(x, W_gcn, deg2[:, :N].reshape(NC, N, 1))
    z = _z_sc(
        y,
        src.reshape(NW, NCH, K),
        dst.reshape(NW, NCH, K),
        w.reshape(NW, NCH, K),
    )
    return _tc_back(
        z, y, dinv, b_gcn.reshape(1, D), W_lin, b_lin.reshape(1, 1)
    )
